# baseline (speedup 1.0000x reference)
import functools

import jax
import jax.numpy as jnp
from jax import lax
from jax.experimental import pallas as pl
from jax.experimental.pallas import tpu as pltpu
from jax.experimental.pallas import tpu_sc as plsc

N = 100000
D = 128
DEG = 3

NC = 2
NS = 16
NW = NC * NS
CHUNK = 3136
NPAD = NW * CHUNK
PW = 16

RSUB = 112
KSUB = CHUNK // RSUB
NBUF = 4


def _sc_gather_body(ns_hbm, px_hbm, py_hbm, pz_hbm, nb0, nb1, nb2,
                    g0, g1, g2, s_out,
                    idx_v, nbuf_v, nbrc_v, own_v, s_v,
                    gsem, ssem, psem):
    wid = lax.axis_index("s") * NC + lax.axis_index("c")
    base = wid * CHUNK
    nbs = (nb0, nb1, nb2)
    gs = (g0, g1, g2)
    pcs = (px_hbm, py_hbm, pz_hbm)
    for j in range(3):
        pltpu.sync_copy(nbs[j].at[pl.ds(base, CHUNK)], idx_v.at[j])

    for c in range(3):
        pltpu.sync_copy(pcs[c].at[pl.ds(base, CHUNK)], own_v.at[c])

    @pl.loop(0, KSUB)
    def _(k):
        sl = pl.ds(k * RSUB, RSUB)
        for j in range(3):
            for c in range(3):
                pltpu.async_copy(pcs[c].at[idx_v.at[j, sl]],
                                 nbrc_v.at[j, c, sl], psem)

    @pl.loop(0, KSUB)
    def _(k):
        for _ in range(9):
            pltpu.make_async_copy(pcs[0].at[idx_v.at[0, pl.ds(0, RSUB)]],
                                  nbrc_v.at[0, 0, pl.ds(0, RSUB)],
                                  psem).wait()

    @pl.loop(0, CHUNK // 16)
    def _(gidx):
        sl = pl.ds(gidx * 16, 16)
        x0, y0, z0 = nbrc_v[0, 0, sl], nbrc_v[0, 1, sl], nbrc_v[0, 2, sl]
        x1, y1, z1 = nbrc_v[1, 0, sl], nbrc_v[1, 1, sl], nbrc_v[1, 2, sl]
        x2, y2, z2 = nbrc_v[2, 0, sl], nbrc_v[2, 1, sl], nbrc_v[2, 2, sl]
        xo, yo, zo = own_v[0, sl], own_v[1, sl], own_v[2, sl]
        third = jnp.float32(1.0 / 3.0)
        cx = (x0 + x1 + x2) * third
        cy = (y0 + y1 + y2) * third
        cz = (z0 + z1 + z2) * third
        ax, ay, az = x1 - x0, y1 - y0, z1 - z0
        bx, by, bz = x2 - x0, y2 - y0, z2 - z0
        nx = ay * bz - az * by
        ny = az * bx - ax * bz
        nz = ax * by - ay * bx
        det = nx * (xo - cx) + ny * (yo - cy) + nz * (zo - cz)
        s_v[sl] = jnp.where(det < 0, jnp.float32(-1.0), jnp.float32(1.0))

    pltpu.sync_copy(s_v, s_out.at[pl.ds(base, CHUNK)])

    def gather(j, k, b):
        sl = idx_v.at[j, pl.ds(k * RSUB, RSUB)]
        pltpu.async_copy(ns_hbm.at[sl], nbuf_v.at[b], gsem.at[b])

    def wait_gather(b):
        pltpu.make_async_copy(ns_hbm.at[idx_v.at[0, pl.ds(0, RSUB)]],
                              nbuf_v.at[b], gsem.at[b]).wait()

    def scatter(g, k, b):
        dst = pl.ds(base + k * RSUB, RSUB)
        pltpu.async_copy(nbuf_v.at[b], g.at[dst], ssem.at[b])

    def wait_scatter(g, b):
        pltpu.make_async_copy(nbuf_v.at[b], g.at[pl.ds(base, RSUB)],
                              ssem.at[b]).wait()

    for j in range(3):
        for b in range(NBUF):
            if j > 0:
                wait_scatter(gs[j - 1], b)
            gather(j, b, b)

        @pl.loop(0, KSUB - NBUF, step=NBUF)
        def _(kk):
            for b in range(NBUF):
                k = kk + b
                wait_gather(b)
                scatter(gs[j], k, b)
                wait_scatter(gs[j], b)
                gather(j, k + NBUF, b)

        for b in range(NBUF):
            wait_gather(b)
            scatter(gs[j], KSUB - NBUF + b, b)
    for b in range(NBUF):
        wait_scatter(gs[2], b)


@functools.cache
def _sc_gather():
    return functools.partial(
        pl.kernel,
        mesh=plsc.VectorSubcoreMesh(core_axis_name="c", subcore_axis_name="s",
                                    num_cores=NC, num_subcores=NS),
        out_type=[jax.ShapeDtypeStruct((NPAD, D), jnp.bfloat16)] * 3
               + [jax.ShapeDtypeStruct((NPAD,), jnp.float32)],
        scratch_types=[
            pltpu.VMEM((3, CHUNK), jnp.int32),
            pltpu.VMEM((NBUF, RSUB, D), jnp.bfloat16),
            pltpu.VMEM((3, 3, CHUNK), jnp.float32),
            pltpu.VMEM((3, CHUNK), jnp.float32),
            pltpu.VMEM((CHUNK,), jnp.float32),
            pltpu.SemaphoreType.DMA((NBUF,)),
            pltpu.SemaphoreType.DMA((NBUF,)),
            pltpu.SemaphoreType.DMA,
        ],
        compiler_params=pltpu.CompilerParams(use_tc_tiling_on_sc=False),
    )(_sc_gather_body)


BLK = 2000


def _tc_body(g0, g1, g2, sref, chib,
             w1t, b1r, w3t, b3r, w4t, b4r, w5t, b5r, w6t, b6r, out):
    s = sref[...]

    def mm(x, w):
        return jax.lax.dot_general(x, w, (((1,), (0,)), ((), ())),
                                   preferred_element_type=jnp.float32)

    w1 = w1t[...].astype(jnp.float32)
    e0 = mm(g0[...].astype(jnp.float32), w1) + b1r[...]
    e1 = mm(g1[...].astype(jnp.float32), w1) + b1r[...]
    e2 = mm(g2[...].astype(jnp.float32), w1) + b1r[...]
    m0 = s * (e2 - e1)
    m1 = s * (e0 - e2)
    m2 = s * (e1 - e0)

    w3 = w3t[...].astype(jnp.float32)
    w4 = w4t[...].astype(jnp.float32)
    w5 = w5t[...].astype(jnp.float32)
    w6 = w6t[...].astype(jnp.float32)
    b3 = b3r[...]
    b4 = b4r[...]
    b5 = b5r[...]
    b6 = b6r[...]

    def silu(x):
        return x * jax.lax.logistic(x)

    acc = chib[...]
    for e, m in ((e0, m0), (e1, m1), (e2, m2)):
        cu = mm(silu(mm(m, w3) + b3), w4) + b4
        h = mm(silu(mm(e + cu, w5) + b5), w6) + b6
        acc = acc + h
    out[...] = acc


def _tc_call(g0, g1, g2, s2d, node_chiral, *wb):
    nblk = N // BLK
    row = lambda i: (i, 0)
    fixed = lambda i: (0, 0)
    gspec = pl.BlockSpec((BLK, D), row)
    sspec = pl.BlockSpec((BLK, 1), row)
    wspec = pl.BlockSpec((D, D), fixed)
    bspec = pl.BlockSpec((1, D), fixed)
    in_specs = [gspec] * 3 + [sspec, gspec]
    for _ in range(5):
        in_specs += [wspec, bspec]
    return pl.pallas_call(
        _tc_body,
        grid=(nblk,),
        in_specs=in_specs,
        out_specs=gspec,
        out_shape=jax.ShapeDtypeStruct((N, D), jnp.float32),
        compiler_params=pltpu.CompilerParams(
            dimension_semantics=("arbitrary",)),
    )(g0, g1, g2, s2d, node_chiral, *wb)


def kernel(node_scalar, node_chiral, edge_index, pos,
           W1, b1, W3, b3, W4, b4, W5, b5, W6, b6):
    nbr = edge_index[:, 1].reshape(N, DEG)
    pad = NPAD - N
    nbrT = jnp.concatenate(
        [nbr.T, jnp.zeros((DEG, pad), jnp.int32)], axis=1)
    posT = jnp.concatenate(
        [pos.T, jnp.zeros((3, pad), jnp.float32)], axis=1)
    g0, g1, g2, s = _sc_gather()(
        node_scalar.astype(jnp.bfloat16), posT[0], posT[1], posT[2],
        nbrT[0], nbrT[1], nbrT[2])
    wb = []
    for W, b in ((W1, b1), (W3, b3), (W4, b4), (W5, b5), (W6, b6)):
        wb += [W.T, b.reshape(1, D)]
    return _tc_call(g0, g1, g2, s.reshape(NPAD, 1), node_chiral, *wb)


# speedup vs baseline: 1.1126x; 1.1126x over previous
import functools

import jax
import jax.numpy as jnp
from jax import lax
from jax.experimental import pallas as pl
from jax.experimental.pallas import tpu as pltpu
from jax.experimental.pallas import tpu_sc as plsc

N = 100000
D = 128
DEG = 3

NC = 2
NS = 16
NW = NC * NS
CHUNK = 3136
NPAD = NW * CHUNK
RSUB = 224
KSUB = CHUNK // RSUB
PW = 16
NBUF = 2


def _sc_gather_body(ns_hbm, pos_hbm, nb0, nb1, nb2,
                    g0, g1, g2, p0, p1, p2,
                    idx_v, rows_v, prow_v, gsem, psem, ssem, qsem):
    wid = lax.axis_index("s") * NC + lax.axis_index("c")
    base = wid * CHUNK

    def gather(k, b):
        sl = idx_v.at[pl.ds(k * RSUB, RSUB)]
        pltpu.async_copy(ns_hbm.at[sl], rows_v.at[b], gsem.at[b])
        pltpu.async_copy(pos_hbm.at[sl], prow_v.at[b], psem.at[b])

    def scatter(k, b, g, p):
        dst = pl.ds(base + k * RSUB, RSUB)
        pltpu.async_copy(rows_v.at[b], g.at[dst], ssem.at[b])
        pltpu.async_copy(prow_v.at[b], p.at[dst], qsem.at[b])

    def wait_gather(b):
        pltpu.make_async_copy(ns_hbm.at[idx_v.at[pl.ds(0, RSUB)]],
                              rows_v.at[b], gsem.at[b]).wait()
        pltpu.make_async_copy(pos_hbm.at[idx_v.at[pl.ds(0, RSUB)]],
                              prow_v.at[b], psem.at[b]).wait()

    def wait_scatter(b, g, p):
        dst = pl.ds(base, RSUB)
        pltpu.make_async_copy(rows_v.at[b], g.at[dst], ssem.at[b]).wait()
        pltpu.make_async_copy(prow_v.at[b], p.at[dst], qsem.at[b]).wait()

    tabs = ((nb0, g0, p0), (nb1, g1, p1), (nb2, g2, p2))
    for j, (nb, g, p) in enumerate(tabs):
        pltpu.sync_copy(nb.at[pl.ds(base, CHUNK)], idx_v)
        for b in range(NBUF):
            if j > 0:
                wait_scatter(b, tabs[j - 1][1], tabs[j - 1][2])
            gather(b, b)

        @pl.loop(0, KSUB - NBUF, step=NBUF)
        def _(kk):
            for b in range(NBUF):
                k = kk + b
                wait_gather(b)
                scatter(k, b, g, p)
                wait_scatter(b, g, p)
                gather(k + NBUF, b)

        for b in range(NBUF):
            wait_gather(b)
            scatter(KSUB - NBUF + b, b, g, p)
    for b in range(NBUF):
        wait_scatter(b, g2, p2)


@functools.cache
def _sc_gather():
    return functools.partial(
        pl.kernel,
        mesh=plsc.VectorSubcoreMesh(core_axis_name="c", subcore_axis_name="s",
                                    num_cores=NC, num_subcores=NS),
        out_type=[jax.ShapeDtypeStruct((NPAD, D), jnp.float32)] * 3
               + [jax.ShapeDtypeStruct((NPAD, PW), jnp.float32)] * 3,
        scratch_types=[
            pltpu.VMEM((CHUNK,), jnp.int32),
            pltpu.VMEM((NBUF, RSUB, D), jnp.float32),
            pltpu.VMEM((NBUF, RSUB, PW), jnp.float32),
            pltpu.SemaphoreType.DMA((NBUF,)),
            pltpu.SemaphoreType.DMA((NBUF,)),
            pltpu.SemaphoreType.DMA((NBUF,)),
            pltpu.SemaphoreType.DMA((NBUF,)),
        ],
        compiler_params=pltpu.CompilerParams(use_tc_tiling_on_sc=False),
    )(_sc_gather_body)


BLK = 2000


def _tc_body(g0, g1, g2, p0, p1, p2, posb, chib,
             w1r, b1r, w3r, b3r, w4r, b4r, w5r, b5r, w6r, b6r, out):
    q0 = p0[...]
    q1 = p1[...]
    q2 = p2[...]
    c = (q0 + q1 + q2) * (1.0 / 3.0)
    a = q1 - q0
    b = q2 - q0
    bv = posb[...] - c

    def comp(x, i):
        return x[:, i:i + 1]

    nx = comp(a, 1) * comp(b, 2) - comp(a, 2) * comp(b, 1)
    ny = comp(a, 2) * comp(b, 0) - comp(a, 0) * comp(b, 2)
    nz = comp(a, 0) * comp(b, 1) - comp(a, 1) * comp(b, 0)
    dot = nx * comp(bv, 0) + ny * comp(bv, 1) + nz * comp(bv, 2)
    s = jnp.where(dot < 0, -1.0, 1.0)

    def mm(x, w):
        return jax.lax.dot_general(x, w, (((1,), (1,)), ((), ())),
                                   preferred_element_type=jnp.float32)

    w1 = w1r[...]
    e0 = mm(g0[...], w1) + b1r[...]
    e1 = mm(g1[...], w1) + b1r[...]
    e2 = mm(g2[...], w1) + b1r[...]
    m0 = s * (e2 - e1)
    m1 = s * (e0 - e2)
    m2 = s * (e1 - e0)

    w3 = w3r[...]
    w4 = w4r[...]
    w5 = w5r[...]
    w6 = w6r[...]
    b3 = b3r[...]
    b4 = b4r[...]
    b5 = b5r[...]
    b6 = b6r[...]

    def silu(x):
        return x * jax.lax.logistic(x)

    acc = chib[...]
    for e, m in ((e0, m0), (e1, m1), (e2, m2)):
        cu = mm(silu(mm(m, w3) + b3), w4) + b4
        h = mm(silu(mm(e + cu, w5) + b5), w6) + b6
        acc = acc + h
    out[...] = acc


def _tc_call(g0, g1, g2, p0, p1, p2, pos_pad, node_chiral, *wb):
    nblk = N // BLK
    row = lambda i: (i, 0)
    fixed = lambda i: (0, 0)
    gspec = pl.BlockSpec((BLK, D), row)
    pspec = pl.BlockSpec((BLK, PW), row)
    wspec = pl.BlockSpec((D, D), fixed)
    bspec = pl.BlockSpec((1, D), fixed)
    in_specs = [gspec] * 3 + [pspec] * 4 + [gspec]
    for _ in range(5):
        in_specs += [wspec, bspec]
    return pl.pallas_call(
        _tc_body,
        grid=(nblk,),
        in_specs=in_specs,
        out_specs=gspec,
        out_shape=jax.ShapeDtypeStruct((N, D), jnp.float32),
        compiler_params=pltpu.CompilerParams(
            dimension_semantics=("arbitrary",)),
    )(g0, g1, g2, p0, p1, p2, pos_pad, node_chiral, *wb)


def kernel(node_scalar, node_chiral, edge_index, pos,
           W1, b1, W3, b3, W4, b4, W5, b5, W6, b6):
    nbr = edge_index[:, 1].reshape(N, DEG)
    pad = NPAD - N
    nbrT = jnp.concatenate(
        [nbr.T, jnp.zeros((DEG, pad), jnp.int32)], axis=1)
    pos_pad = jnp.pad(pos, ((0, 0), (0, PW - 3)))
    g0, g1, g2, p0, p1, p2 = _sc_gather()(
        node_scalar, pos_pad, nbrT[0], nbrT[1], nbrT[2])
    wb = []
    for W, b in ((W1, b1), (W3, b3), (W4, b4), (W5, b5), (W6, b6)):
        wb += [W, b.reshape(1, D)]
    return _tc_call(g0, g1, g2, p0, p1, p2, pos_pad, node_chiral, *wb)
